# baseline (speedup 1.0000x reference)
import functools

import jax
import jax.numpy as jnp
from jax import lax
from jax.experimental import pallas as pl
from jax.experimental.pallas import tpu as pltpu
from jax.experimental.pallas import tpu_sc as plsc

B = 50000
EMB = 64
DFEAT = 128

_INFO = plsc.get_sparse_core_info()
_NC = _INFO.num_cores
_NS = _INFO.num_subcores
_NW = _NC * _NS

CHUNK = 80
_BLK = 2048
_NBLK = -(-B // _BLK)
_BP = _NBLK * _BLK
_USLAB = _BP // _NW
_UCH = _USLAB // CHUNK

_NCHUNKS = B // CHUNK
_MAXC = -(-_NCHUNKS // _NW)
_FULL = _NCHUNKS - (_MAXC - 1) * _NW
_SLAB = _MAXC * CHUNK
_NID_PAD = (_NW - 1) * (_MAXC - 1) * CHUNK + _FULL * CHUNK + _SLAB - B


_NBUF = 4


def _sc_user_body(emb_hbm, nidu_hbm, xu_hbm, idxu_v, *bufs_sems):
    bufs, sems = bufs_sems[:_NBUF], bufs_sems[_NBUF:]
    wid = lax.axis_index("s") * _NC + lax.axis_index("c")
    ubase = wid * _USLAB
    pltpu.sync_copy(nidu_hbm.at[pl.ds(ubase, _USLAB)], idxu_v)
    cps = [None] * _NBUF
    for j in range(_UCH + _NBUF - 1):
        if j < _UCH:
            cps[j % _NBUF] = pltpu.async_copy(
                emb_hbm.at[idxu_v.at[pl.ds(j * CHUNK, CHUNK)]],
                bufs[j % _NBUF], sems[j % _NBUF])
        d = j - (_NBUF - 1)
        if 0 <= d < _UCH:
            cps[d % _NBUF].wait()
            pltpu.sync_copy(bufs[d % _NBUF],
                            xu_hbm.at[pl.ds(ubase + d * CHUNK, CHUNK)])


def _sc_item_body(feats_hbm, nidi_hbm, rows_hbm, idxi_v, *bufs_sems):
    bufs, sems = bufs_sems[:_NBUF], bufs_sems[_NBUF:]
    wid = lax.axis_index("s") * _NC + lax.axis_index("c")
    nchunks = jnp.where(wid < _FULL, _MAXC, _MAXC - 1)
    ibase = wid * ((_MAXC - 1) * CHUNK) + jnp.minimum(wid, _FULL) * CHUNK
    pltpu.sync_copy(nidi_hbm.at[pl.ds(ibase, _SLAB)], idxi_v)
    cps = [None] * _NBUF
    for j in range(_MAXC + _NBUF - 1):
        if j < _MAXC:
            cps[j % _NBUF] = pltpu.async_copy(
                feats_hbm.at[idxi_v.at[pl.ds(j * CHUNK, CHUNK)]],
                bufs[j % _NBUF], sems[j % _NBUF])
        d = j - (_NBUF - 1)
        if 0 <= d < _MAXC:
            cps[d % _NBUF].wait()

            @pl.when(d < nchunks)
            def _():
                pltpu.sync_copy(
                    bufs[d % _NBUF],
                    rows_hbm.at[pl.ds(ibase + d * CHUNK, CHUNK)])


_MESH = plsc.VectorSubcoreMesh(core_axis_name="c", subcore_axis_name="s")
_PARAMS = pltpu.CompilerParams(use_tc_tiling_on_sc=False)

_sc_user = functools.partial(
    pl.kernel,
    mesh=_MESH,
    out_type=[jax.ShapeDtypeStruct((_BP, EMB), jnp.float32)],
    scratch_types=(
        [pltpu.VMEM((_USLAB,), jnp.int32)]
        + [pltpu.VMEM((CHUNK, EMB), jnp.float32)] * _NBUF
        + [pltpu.SemaphoreType.DMA] * _NBUF
    ),
    compiler_params=_PARAMS,
)(_sc_user_body)

_sc_item = functools.partial(
    pl.kernel,
    mesh=_MESH,
    out_type=[jax.ShapeDtypeStruct((B, DFEAT), jnp.float32)],
    scratch_types=(
        [pltpu.VMEM((_SLAB,), jnp.int32)]
        + [pltpu.VMEM((CHUNK, DFEAT), jnp.float32)] * _NBUF
        + [pltpu.SemaphoreType.DMA] * _NBUF
    ),
    compiler_params=_PARAMS,
)(_sc_item_body)


def _tc_mm_body(wt_ref, rows_ref, xiT_ref):
    xiT_ref[...] = lax.dot_general(
        wt_ref[...], rows_ref[...],
        dimension_numbers=(((1,), (1,)), ((), ())),
        preferred_element_type=jnp.float32)


def _tc_mm(wt, rows):
    return pl.pallas_call(
        _tc_mm_body,
        grid=(_NBLK,),
        in_specs=[
            pl.BlockSpec((EMB, DFEAT), lambda i: (0, 0)),
            pl.BlockSpec((_BLK, DFEAT), lambda i: (i, 0)),
        ],
        out_specs=pl.BlockSpec((EMB, _BLK), lambda i: (0, i)),
        out_shape=jax.ShapeDtypeStruct((EMB, B), jnp.float32),
    )(wt, rows)


def _tc_unpack_body(xu_ref, xuT_ref):
    pt = jnp.transpose(xu_ref[...])
    xuT_ref[...] = jnp.concatenate([pt[:EMB], pt[EMB:]], axis=1)


def _tc_unpack(xu_pairs):
    return pl.pallas_call(
        _tc_unpack_body,
        grid=(_NBLK,),
        in_specs=[pl.BlockSpec((_BLK // 2, 2 * EMB), lambda i: (i, 0))],
        out_specs=pl.BlockSpec((EMB, _BLK), lambda i: (0, i)),
        out_shape=jax.ShapeDtypeStruct((EMB, B), jnp.float32),
    )(xu_pairs)


def kernel(emb_user, feats_item, W_item, nid_user, nid_item):
    nid_u = jnp.pad(nid_user.astype(jnp.int32), (0, _BP - B))
    nid_perm = nid_u.reshape(_NBLK, 2, _BLK // 2).transpose(0, 2, 1).reshape(-1)
    nid_i = jnp.pad(nid_item.astype(jnp.int32), (0, _NID_PAD))
    (rows,) = _sc_item(feats_item, nid_i)
    (xu_pairs,) = _sc_user(emb_user, nid_perm)
    xu_pairs = xu_pairs.reshape(_BP // 2, 2 * EMB)
    x_itemT = _tc_mm(W_item.T, rows)
    x_userT = _tc_unpack(xu_pairs)
    return (x_userT.T, x_itemT.T)


# speedup vs baseline: 1.0343x; 1.0343x over previous
import functools

import jax
import jax.numpy as jnp
from jax import lax
from jax.experimental import pallas as pl
from jax.experimental.pallas import tpu as pltpu
from jax.experimental.pallas import tpu_sc as plsc

B = 50000
EMB = 64
DFEAT = 128

_INFO = plsc.get_sparse_core_info()
_NC = _INFO.num_cores
_NS = _INFO.num_subcores
_NW = _NC * _NS

CHUNK = 80
_BLK = 2048
_NBLK = -(-B // _BLK)
_BP = _NBLK * _BLK
_USLAB = _BP // _NW
_UCH = _USLAB // CHUNK

_NCHUNKS = B // CHUNK
_MAXC = -(-_NCHUNKS // _NW)
_FULL = _NCHUNKS - (_MAXC - 1) * _NW
_SLAB = _MAXC * CHUNK
_NID_PAD = (_NW - 1) * (_MAXC - 1) * CHUNK + _FULL * CHUNK + _SLAB - B


_NBUF = 4


_UCHUNK = 128
_UFULL = _USLAB // _UCHUNK
_UTAIL = _USLAB - _UFULL * _UCHUNK


def _sc_user_body(emb_hbm, nidu_hbm, xu_hbm, idxu_v, tbuf_v, *bufs_sems):
    bufs, sems = bufs_sems[:_NBUF], bufs_sems[_NBUF:]
    wid = lax.axis_index("s") * _NC + lax.axis_index("c")
    ubase = wid * _USLAB
    pltpu.sync_copy(nidu_hbm.at[pl.ds(ubase, _USLAB)], idxu_v)
    ct = pltpu.async_copy(
        emb_hbm.at[idxu_v.at[pl.ds(_UFULL * _UCHUNK, _UTAIL)]],
        tbuf_v, sems[_NBUF])
    cps = [None] * _NBUF
    for j in range(_UFULL + _NBUF - 1):
        if j < _UFULL:
            cps[j % _NBUF] = pltpu.async_copy(
                emb_hbm.at[idxu_v.at[pl.ds(j * _UCHUNK, _UCHUNK)]],
                bufs[j % _NBUF], sems[j % _NBUF])
        d = j - (_NBUF - 1)
        if 0 <= d < _UFULL:
            cps[d % _NBUF].wait()
            pltpu.sync_copy(bufs[d % _NBUF],
                            xu_hbm.at[pl.ds(ubase + d * _UCHUNK, _UCHUNK)])
    ct.wait()
    pltpu.sync_copy(tbuf_v,
                    xu_hbm.at[pl.ds(ubase + _UFULL * _UCHUNK, _UTAIL)])


def _sc_item_body(feats_hbm, nidi_hbm, rows_hbm, idxi_v, *bufs_sems):
    bufs, sems = bufs_sems[:_NBUF], bufs_sems[_NBUF:]
    wid = lax.axis_index("s") * _NC + lax.axis_index("c")
    nchunks = jnp.where(wid < _FULL, _MAXC, _MAXC - 1)
    ibase = wid * ((_MAXC - 1) * CHUNK) + jnp.minimum(wid, _FULL) * CHUNK
    pltpu.sync_copy(nidi_hbm.at[pl.ds(ibase, _SLAB)], idxi_v)
    cps = [None] * _NBUF
    for j in range(_MAXC + _NBUF - 1):
        if j < _MAXC:
            cps[j % _NBUF] = pltpu.async_copy(
                feats_hbm.at[idxi_v.at[pl.ds(j * CHUNK, CHUNK)]],
                bufs[j % _NBUF], sems[j % _NBUF])
        d = j - (_NBUF - 1)
        if 0 <= d < _MAXC:
            cps[d % _NBUF].wait()

            @pl.when(d < nchunks)
            def _():
                pltpu.sync_copy(
                    bufs[d % _NBUF],
                    rows_hbm.at[pl.ds(ibase + d * CHUNK, CHUNK)])


_MESH = plsc.VectorSubcoreMesh(core_axis_name="c", subcore_axis_name="s")
_PARAMS = pltpu.CompilerParams(use_tc_tiling_on_sc=False)

_sc_user = functools.partial(
    pl.kernel,
    mesh=_MESH,
    out_type=[jax.ShapeDtypeStruct((_BP, EMB), jnp.float32)],
    scratch_types=(
        [pltpu.VMEM((_USLAB,), jnp.int32),
         pltpu.VMEM((_UTAIL, EMB), jnp.float32)]
        + [pltpu.VMEM((_UCHUNK, EMB), jnp.float32)] * _NBUF
        + [pltpu.SemaphoreType.DMA] * (_NBUF + 1)
    ),
    compiler_params=_PARAMS,
)(_sc_user_body)

_sc_item = functools.partial(
    pl.kernel,
    mesh=_MESH,
    out_type=[jax.ShapeDtypeStruct((B, DFEAT), jnp.float32)],
    scratch_types=(
        [pltpu.VMEM((_SLAB,), jnp.int32)]
        + [pltpu.VMEM((CHUNK, DFEAT), jnp.float32)] * _NBUF
        + [pltpu.SemaphoreType.DMA] * _NBUF
    ),
    compiler_params=_PARAMS,
)(_sc_item_body)


def _tc_body(xu_ref, wt_ref, rows_ref, xuT_ref, xiT_ref):
    pt = jnp.transpose(xu_ref[...])
    xuT_ref[...] = jnp.concatenate([pt[:EMB], pt[EMB:]], axis=1)
    xiT_ref[...] = lax.dot_general(
        wt_ref[...], rows_ref[...],
        dimension_numbers=(((1,), (1,)), ((), ())),
        preferred_element_type=jnp.float32)


def _tc_finish(xu_pairs, wt, rows):
    return pl.pallas_call(
        _tc_body,
        grid=(_NBLK,),
        in_specs=[
            pl.BlockSpec((_BLK // 2, 2 * EMB), lambda i: (i, 0)),
            pl.BlockSpec((EMB, DFEAT), lambda i: (0, 0)),
            pl.BlockSpec((_BLK, DFEAT), lambda i: (i, 0)),
        ],
        out_specs=[
            pl.BlockSpec((EMB, _BLK), lambda i: (0, i)),
            pl.BlockSpec((EMB, _BLK), lambda i: (0, i)),
        ],
        out_shape=[
            jax.ShapeDtypeStruct((EMB, B), jnp.float32),
            jax.ShapeDtypeStruct((EMB, B), jnp.float32),
        ],
    )(xu_pairs, wt, rows)


def kernel(emb_user, feats_item, W_item, nid_user, nid_item):
    nid_u = jnp.pad(nid_user.astype(jnp.int32), (0, _BP - B))
    nid_perm = nid_u.reshape(_NBLK, 2, _BLK // 2).transpose(0, 2, 1).reshape(-1)
    nid_i = jnp.pad(nid_item.astype(jnp.int32), (0, _NID_PAD))
    (rows,) = _sc_item(feats_item, nid_i)
    (xu_pairs,) = _sc_user(emb_user, nid_perm)
    xu_pairs = xu_pairs.reshape(_BP // 2, 2 * EMB)
    x_userT, x_itemT = _tc_finish(xu_pairs, W_item.T, rows)
    return (x_userT.T, x_itemT.T)
